# Initial kernel scaffold; baseline (speedup 1.0000x reference)
import functools

import jax
import jax.numpy as jnp
from jax import lax
from jax.experimental import pallas as pl
from jax.experimental.pallas import tpu as pltpu
from jax.experimental.pallas import tpu_sc as plsc

N = 100000
K = 27
M = 50000
CIN = 16
COUT = 16

NC = 2
NS = 16
NW = NC * NS

CHUNK = 128
TOTAL = K * M
PER_W = -(-TOTAL // (NW * CHUNK)) * CHUNK
N_CHUNKS = PER_W // CHUNK
TOTAL_PAD = PER_W * NW

N_PAD = 100096
ROWS_PT = N_PAD // NS

_TC_BLK = 10000


def _tf_body(f_ref, w_ref, o_ref):
    o_ref[...] = jnp.dot(f_ref[...], w_ref[0], preferred_element_type=jnp.float32)


def _transform(features, weights):
    nblk = N // _TC_BLK
    return pl.pallas_call(
        _tf_body,
        grid=(nblk, K),
        in_specs=[
            pl.BlockSpec((_TC_BLK, CIN), lambda r, k: (r, 0)),
            pl.BlockSpec((1, CIN, COUT), lambda r, k: (k, 0, 0)),
        ],
        out_specs=pl.BlockSpec((_TC_BLK, COUT), lambda r, k: (k * nblk + r, 0)),
        out_shape=jax.ShapeDtypeStruct((K * N, COUT), jnp.float32),
    )(features, weights)


def _sc_body(tf_hbm, inidx_hbm, outidx_hbm, zeros_hbm, out_hbm,
             inidx_v, outidx_v, rows_v, acc_sh, sem0, sem1):
    c = lax.axis_index("c")
    s = lax.axis_index("s")
    wid = c * NS + s

    pltpu.sync_copy(inidx_hbm.at[wid], inidx_v)
    pltpu.sync_copy(outidx_hbm.at[wid], outidx_v)

    pltpu.sync_copy(zeros_hbm.at[pl.ds(s * ROWS_PT, ROWS_PT)],
                    acc_sh.at[pl.ds(s * ROWS_PT, ROWS_PT)])
    plsc.subcore_barrier()

    pltpu.async_copy(tf_hbm.at[inidx_v.at[0]], rows_v.at[0], sem0)

    def body(i, carry):
        j0 = 2 * i
        cp1 = pltpu.async_copy(tf_hbm.at[inidx_v.at[j0 + 1]], rows_v.at[1], sem1)
        pltpu.make_async_copy(tf_hbm.at[inidx_v.at[j0]], rows_v.at[0], sem0).wait()
        pltpu.sync_copy(rows_v.at[0], acc_sh.at[outidx_v.at[j0]], add=True)

        @pl.when(j0 + 2 < N_CHUNKS)
        def _():
            pltpu.async_copy(tf_hbm.at[inidx_v.at[j0 + 2]], rows_v.at[0], sem0)

        cp1.wait()
        pltpu.sync_copy(rows_v.at[1], acc_sh.at[outidx_v.at[j0 + 1]], add=True)
        return carry

    lax.fori_loop(0, N_CHUNKS // 2, body, 0)

    plsc.subcore_barrier()
    pltpu.sync_copy(acc_sh.at[pl.ds(s * ROWS_PT, ROWS_PT)],
                    out_hbm.at[c].at[pl.ds(s * ROWS_PT, ROWS_PT)])


_sc_gather_scatter = functools.partial(
    pl.kernel,
    out_type=jax.ShapeDtypeStruct((NC, N_PAD, COUT), jnp.float32),
    mesh=plsc.VectorSubcoreMesh(core_axis_name="c", subcore_axis_name="s",
                                num_cores=NC, num_subcores=NS),
    scratch_types=[
        pltpu.VMEM((N_CHUNKS, CHUNK), jnp.int32),
        pltpu.VMEM((N_CHUNKS, CHUNK), jnp.int32),
        pltpu.VMEM((2, CHUNK, COUT), jnp.float32),
        pltpu.VMEM_SHARED((N_PAD, COUT), jnp.float32),
        pltpu.SemaphoreType.DMA,
        pltpu.SemaphoreType.DMA,
    ],
)(_sc_body)


def _comb_body(a_ref, o_ref):
    o_ref[...] = a_ref[0] + a_ref[1]


def _combine(acc):
    nblk = N // _TC_BLK
    return pl.pallas_call(
        _comb_body,
        grid=(nblk,),
        in_specs=[pl.BlockSpec((NC, _TC_BLK, COUT), lambda r: (0, r, 0))],
        out_specs=pl.BlockSpec((_TC_BLK, COUT), lambda r: (r, 0)),
        out_shape=jax.ShapeDtypeStruct((N, COUT), jnp.float32),
    )(acc)


def kernel(features, kernel, in_idx, out_idx):
    offs = (jnp.arange(K, dtype=jnp.int32) * N)[:, None]
    in_flat = (in_idx + offs).reshape(-1)
    out_flat = out_idx.reshape(-1)
    pad = TOTAL_PAD - TOTAL
    in_flat = jnp.concatenate([in_flat, jnp.zeros((pad,), jnp.int32)])
    out_flat = jnp.concatenate([out_flat, jnp.full((pad,), N, jnp.int32)])
    in_resh = in_flat.reshape(NW, N_CHUNKS, CHUNK)
    out_resh = out_flat.reshape(NW, N_CHUNKS, CHUNK)

    tf = _transform(features, kernel)
    zeros = jnp.zeros((N_PAD, COUT), jnp.float32)
    acc = _sc_gather_scatter(tf, in_resh, out_resh, zeros)
    return _combine(acc)


# speedup vs baseline: 4.9206x; 4.9206x over previous
import functools

import jax
import jax.numpy as jnp
from jax import lax
from jax.experimental import pallas as pl
from jax.experimental.pallas import tpu as pltpu
from jax.experimental.pallas import tpu_sc as plsc

N = 100000
K = 27
M = 50000
CIN = 16
COUT = 16

NC = 2
NS = 16
NW = NC * NS

CHUNK = 128
G = 24
NGROUPS = 14
N_CHUNKS = G * NGROUPS
PER_W = N_CHUNKS * CHUNK
TOTAL = K * M
TOTAL_PAD = PER_W * NW

N_PAD = 100096
ROWS_PT = N_PAD // NS

_TC_BLK = 10000


def _tf_body(f_ref, w_ref, o_ref):
    o_ref[...] = jnp.dot(f_ref[...], w_ref[0], preferred_element_type=jnp.float32)


def _transform(features, weights):
    nblk = N // _TC_BLK
    return pl.pallas_call(
        _tf_body,
        grid=(nblk, K),
        in_specs=[
            pl.BlockSpec((_TC_BLK, CIN), lambda r, k: (r, 0)),
            pl.BlockSpec((1, CIN, COUT), lambda r, k: (k, 0, 0)),
        ],
        out_specs=pl.BlockSpec((_TC_BLK, COUT), lambda r, k: (k * nblk + r, 0)),
        out_shape=jax.ShapeDtypeStruct((K * N, COUT), jnp.float32),
    )(features, weights)


def _sc_body(tf_hbm, inidx_hbm, outidx_hbm, zeros_hbm, out_hbm,
             ibuf, obuf, rows_v, acc_sh, sem0, sem1, semii, semio):
    c = lax.axis_index("c")
    s = lax.axis_index("s")
    wid = c * NS + s

    pltpu.sync_copy(zeros_hbm.at[pl.ds(s * ROWS_PT, ROWS_PT)],
                    acc_sh.at[pl.ds(s * ROWS_PT, ROWS_PT)])
    plsc.subcore_barrier()

    pltpu.sync_copy(inidx_hbm.at[wid].at[0], ibuf.at[0])
    pltpu.sync_copy(outidx_hbm.at[wid].at[0], obuf.at[0])

    def process_group(g, bg):
        @pl.when(g + 1 < NGROUPS)
        def _():
            pltpu.async_copy(inidx_hbm.at[wid].at[g + 1], ibuf.at[1 - bg], semii)
            pltpu.async_copy(outidx_hbm.at[wid].at[g + 1], obuf.at[1 - bg], semio)

        ibuf_g = ibuf.at[bg]
        obuf_g = obuf.at[bg]
        pltpu.async_copy(tf_hbm.at[ibuf_g.at[0]], rows_v.at[0], sem0)

        def inner(i, carry):
            l0 = 2 * i
            cp1 = pltpu.async_copy(tf_hbm.at[ibuf_g.at[l0 + 1]], rows_v.at[1], sem1)
            pltpu.make_async_copy(tf_hbm.at[ibuf_g.at[l0]], rows_v.at[0], sem0).wait()
            pltpu.sync_copy(rows_v.at[0], acc_sh.at[obuf_g.at[l0]], add=True)

            @pl.when(l0 + 2 < G)
            def _():
                pltpu.async_copy(tf_hbm.at[ibuf_g.at[l0 + 2]], rows_v.at[0], sem0)

            cp1.wait()
            pltpu.sync_copy(rows_v.at[1], acc_sh.at[obuf_g.at[l0 + 1]], add=True)
            return carry

        lax.fori_loop(0, G // 2, inner, 0)

        @pl.when(g + 1 < NGROUPS)
        def _():
            pltpu.make_async_copy(inidx_hbm.at[wid].at[g + 1], ibuf.at[1 - bg],
                                  semii).wait()
            pltpu.make_async_copy(outidx_hbm.at[wid].at[g + 1], obuf.at[1 - bg],
                                  semio).wait()

    def outer(g2, carry):
        process_group(2 * g2, 0)
        process_group(2 * g2 + 1, 1)
        return carry

    lax.fori_loop(0, NGROUPS // 2, outer, 0)

    plsc.subcore_barrier()
    pltpu.sync_copy(acc_sh.at[pl.ds(s * ROWS_PT, ROWS_PT)],
                    out_hbm.at[c].at[pl.ds(s * ROWS_PT, ROWS_PT)])


_sc_gather_scatter = functools.partial(
    pl.kernel,
    out_type=jax.ShapeDtypeStruct((NC, N_PAD, COUT), jnp.float32),
    mesh=plsc.VectorSubcoreMesh(core_axis_name="c", subcore_axis_name="s",
                                num_cores=NC, num_subcores=NS),
    scratch_types=[
        pltpu.VMEM((2, G, CHUNK), jnp.int32),
        pltpu.VMEM((2, G, CHUNK), jnp.int32),
        pltpu.VMEM((2, CHUNK, COUT), jnp.float32),
        pltpu.VMEM_SHARED((N_PAD, COUT), jnp.float32),
        pltpu.SemaphoreType.DMA,
        pltpu.SemaphoreType.DMA,
        pltpu.SemaphoreType.DMA,
        pltpu.SemaphoreType.DMA,
    ],
    compiler_params=pltpu.CompilerParams(use_tc_tiling_on_sc=False),
)(_sc_body)


def _comb_body(a_ref, o_ref):
    o_ref[...] = a_ref[0] + a_ref[1]


def _combine(acc):
    nblk = N // _TC_BLK
    return pl.pallas_call(
        _comb_body,
        grid=(nblk,),
        in_specs=[pl.BlockSpec((NC, _TC_BLK, COUT), lambda r: (0, r, 0))],
        out_specs=pl.BlockSpec((_TC_BLK, COUT), lambda r: (r, 0)),
        out_shape=jax.ShapeDtypeStruct((N, COUT), jnp.float32),
    )(acc)


def kernel(features, kernel, in_idx, out_idx):
    offs = (jnp.arange(K, dtype=jnp.int32) * N)[:, None]
    in_flat = (in_idx + offs).reshape(-1)
    out_flat = out_idx.reshape(-1)
    pad = TOTAL_PAD - TOTAL
    in_flat = jnp.concatenate([in_flat, jnp.zeros((pad,), jnp.int32)])
    out_flat = jnp.concatenate([out_flat, jnp.full((pad,), N, jnp.int32)])
    in_resh = in_flat.reshape(NW, NGROUPS, G, CHUNK)
    out_resh = out_flat.reshape(NW, NGROUPS, G, CHUNK)

    tf = _transform(features, kernel)
    zeros = jnp.zeros((N_PAD, COUT), jnp.float32)
    acc = _sc_gather_scatter(tf, in_resh, out_resh, zeros)
    return _combine(acc)


# speedup vs baseline: 13.4245x; 2.7282x over previous
import functools

import jax
import jax.numpy as jnp
from jax import lax
from jax.experimental import pallas as pl
from jax.experimental.pallas import tpu as pltpu
from jax.experimental.pallas import tpu_sc as plsc

N = 100000
K = 27
M = 50000
CIN = 16
COUT = 16

NC = 2
NS = 16
NW = NC * NS

CHUNK = 128
G = 24
NGROUPS = 14
N_CHUNKS = G * NGROUPS
PER_W = N_CHUNKS * CHUNK
TOTAL = K * M
TOTAL_PAD = PER_W * NW

N_PAD = 100096
ROWS_PT = N_PAD // NS

NP8 = 100032
R8 = NP8 // 8

_TC_BLK = 10000


def _tf_body(f_ref, w_ref, o_ref):
    o_ref[...] = jnp.dot(f_ref[...], w_ref[0], preferred_element_type=jnp.float32)


def _transform(features8, wbd):
    return pl.pallas_call(
        _tf_body,
        grid=(K,),
        in_specs=[
            pl.BlockSpec((R8, 8 * CIN), lambda k: (0, 0)),
            pl.BlockSpec((1, 8 * CIN, 8 * COUT), lambda k: (k, 0, 0)),
        ],
        out_specs=pl.BlockSpec((R8, 8 * COUT), lambda k: (k, 0)),
        out_shape=jax.ShapeDtypeStruct((K * R8, 8 * COUT), jnp.float32),
    )(features8, wbd)


def _sc_body(tf_hbm, inidx_hbm, outidx_hbm, zeros_hbm, out_hbm,
             ibuf, obuf, rows_v, acc_sh, sem0, sem1, semii, semio):
    c = lax.axis_index("c")
    s = lax.axis_index("s")
    wid = c * NS + s

    pltpu.sync_copy(zeros_hbm.at[pl.ds(s * ROWS_PT, ROWS_PT)],
                    acc_sh.at[pl.ds(s * ROWS_PT, ROWS_PT)])
    plsc.subcore_barrier()

    pltpu.sync_copy(inidx_hbm.at[wid].at[0], ibuf.at[0])
    pltpu.sync_copy(outidx_hbm.at[wid].at[0], obuf.at[0])

    def process_group(g, bg):
        @pl.when(g + 1 < NGROUPS)
        def _():
            pltpu.async_copy(inidx_hbm.at[wid].at[g + 1], ibuf.at[1 - bg], semii)
            pltpu.async_copy(outidx_hbm.at[wid].at[g + 1], obuf.at[1 - bg], semio)

        ibuf_g = ibuf.at[bg]
        obuf_g = obuf.at[bg]
        pltpu.async_copy(tf_hbm.at[ibuf_g.at[0]], rows_v.at[0], sem0)

        def inner(i, carry):
            l0 = 2 * i
            cp1 = pltpu.async_copy(tf_hbm.at[ibuf_g.at[l0 + 1]], rows_v.at[1], sem1)
            pltpu.make_async_copy(tf_hbm.at[ibuf_g.at[l0]], rows_v.at[0], sem0).wait()
            pltpu.sync_copy(rows_v.at[0], acc_sh.at[obuf_g.at[l0]], add=True)

            @pl.when(l0 + 2 < G)
            def _():
                pltpu.async_copy(tf_hbm.at[ibuf_g.at[l0 + 2]], rows_v.at[0], sem0)

            cp1.wait()
            pltpu.sync_copy(rows_v.at[1], acc_sh.at[obuf_g.at[l0 + 1]], add=True)
            return carry

        lax.fori_loop(0, G // 2, inner, 0)

        @pl.when(g + 1 < NGROUPS)
        def _():
            pltpu.make_async_copy(inidx_hbm.at[wid].at[g + 1], ibuf.at[1 - bg],
                                  semii).wait()
            pltpu.make_async_copy(outidx_hbm.at[wid].at[g + 1], obuf.at[1 - bg],
                                  semio).wait()

    def outer(g2, carry):
        process_group(2 * g2, 0)
        process_group(2 * g2 + 1, 1)
        return carry

    lax.fori_loop(0, NGROUPS // 2, outer, 0)

    plsc.subcore_barrier()
    pltpu.sync_copy(acc_sh.at[pl.ds(s * ROWS_PT, ROWS_PT)],
                    out_hbm.at[c].at[pl.ds(s * ROWS_PT, ROWS_PT)])


_sc_gather_scatter = functools.partial(
    pl.kernel,
    out_type=jax.ShapeDtypeStruct((NC, N_PAD, COUT), jnp.float32),
    mesh=plsc.VectorSubcoreMesh(core_axis_name="c", subcore_axis_name="s",
                                num_cores=NC, num_subcores=NS),
    scratch_types=[
        pltpu.VMEM((2, G, CHUNK), jnp.int32),
        pltpu.VMEM((2, G, CHUNK), jnp.int32),
        pltpu.VMEM((2, CHUNK, COUT), jnp.float32),
        pltpu.VMEM_SHARED((N_PAD, COUT), jnp.float32),
        pltpu.SemaphoreType.DMA,
        pltpu.SemaphoreType.DMA,
        pltpu.SemaphoreType.DMA,
        pltpu.SemaphoreType.DMA,
    ],
    compiler_params=pltpu.CompilerParams(use_tc_tiling_on_sc=False),
)(_sc_body)


def _comb_body(a_ref, o_ref):
    o_ref[...] = a_ref[0] + a_ref[1]


def _combine(acc):
    nblk = N // _TC_BLK
    return pl.pallas_call(
        _comb_body,
        grid=(nblk,),
        in_specs=[pl.BlockSpec((NC, _TC_BLK, COUT), lambda r: (0, r, 0))],
        out_specs=pl.BlockSpec((_TC_BLK, COUT), lambda r: (r, 0)),
        out_shape=jax.ShapeDtypeStruct((N, COUT), jnp.float32),
    )(acc)


def kernel(features, kernel, in_idx, out_idx):
    offs = (jnp.arange(K, dtype=jnp.int32) * NP8)[:, None]
    in_flat = (in_idx + offs).reshape(-1)
    out_flat = out_idx.reshape(-1)
    pad = TOTAL_PAD - TOTAL
    in_flat = jnp.concatenate([in_flat, jnp.zeros((pad,), jnp.int32)])
    out_flat = jnp.concatenate([out_flat, jnp.full((pad,), N, jnp.int32)])
    in_resh = in_flat.reshape(NW, NGROUPS, G, CHUNK)
    out_resh = out_flat.reshape(NW, NGROUPS, G, CHUNK)

    features8 = jnp.pad(features, ((0, NP8 - N), (0, 0))).reshape(R8, 8 * CIN)
    wbd = jnp.einsum('qr,kcd->kqcrd', jnp.eye(8, dtype=jnp.float32),
                     kernel).reshape(K, 8 * CIN, 8 * COUT)

    tf = _transform(features8, wbd).reshape(K * NP8, COUT)
    zeros = jnp.zeros((N_PAD, COUT), jnp.float32)
    acc = _sc_gather_scatter(tf, in_resh, out_resh, zeros)
    return _combine(acc)


# speedup vs baseline: 18.1123x; 1.3492x over previous
import functools

import jax
import jax.numpy as jnp
from jax import lax
from jax.experimental import pallas as pl
from jax.experimental.pallas import tpu as pltpu
from jax.experimental.pallas import tpu_sc as plsc

N = 100000
K = 27
M = 50000
CIN = 16
COUT = 16

NC = 2
NS = 16
NW = NC * NS

CHUNK = 128
G = 24
NGROUPS = 14
N_CHUNKS = G * NGROUPS
PER_W = N_CHUNKS * CHUNK
TOTAL = K * M
TOTAL_PAD = PER_W * NW
NBUF = 6

N_PAD = 100096
ROWS_PT = N_PAD // NS

NP8 = 100032
R8 = NP8 // 8

_CB_BLK = 3128


def _tf_body(f_ref, w_ref, o_ref):
    o_ref[...] = jnp.dot(f_ref[...], w_ref[0], preferred_element_type=jnp.float32)


def _transform(features8, wbd):
    return pl.pallas_call(
        _tf_body,
        grid=(K,),
        in_specs=[
            pl.BlockSpec((R8, 8 * CIN), lambda k: (0, 0)),
            pl.BlockSpec((1, 8 * CIN, 8 * COUT), lambda k: (k, 0, 0)),
        ],
        out_specs=pl.BlockSpec((R8, 8 * COUT), lambda k: (k, 0)),
        out_shape=jax.ShapeDtypeStruct((K * R8, 8 * COUT), jnp.float32),
    )(features8, wbd)


def _sc_body(tf_hbm, inidx_hbm, outidx_hbm, out_hbm,
             ibuf, obuf, rows_v, acc_sh, *sems):
    semg = sems[0:NBUF]
    sems_ = sems[NBUF:2 * NBUF]
    semii, semio = sems[2 * NBUF], sems[2 * NBUF + 1]
    c = lax.axis_index("c")
    s = lax.axis_index("s")
    wid = c * NS + s
    base = s * ROWS_PT

    pltpu.async_copy(inidx_hbm.at[wid].at[0], ibuf.at[0], semii)
    pltpu.async_copy(outidx_hbm.at[wid].at[0], obuf.at[0], semio)

    zv = jnp.zeros((COUT,), jnp.float32)

    def zstore(i, carry):
        rows_v[0, i, :] = zv
        return carry

    lax.fori_loop(0, CHUNK, zstore, 0)
    for z in range(48):
        pltpu.async_copy(rows_v.at[0],
                         acc_sh.at[pl.ds(base + z * CHUNK, CHUNK)], sems_[0])
    pltpu.async_copy(rows_v.at[0].at[pl.ds(0, 112)],
                     acc_sh.at[pl.ds(base + 48 * CHUNK, 112)], sems_[1])
    pltpu.make_async_copy(tf_hbm.at[pl.ds(0, 48 * CHUNK)],
                          acc_sh.at[pl.ds(base, 48 * CHUNK)], sems_[0]).wait()
    pltpu.make_async_copy(tf_hbm.at[pl.ds(0, 112)],
                          acc_sh.at[pl.ds(base + 48 * CHUNK, 112)], sems_[1]).wait()
    plsc.subcore_barrier()

    def step(t, carry):
        for b in range(NBUF):
            j = NBUF * t + b
            g = j // G
            bg = g % 2
            l = j % G

            @pl.when(l == 0)
            def _():
                pltpu.make_async_copy(inidx_hbm.at[wid].at[g], ibuf.at[bg],
                                      semii).wait()
                pltpu.make_async_copy(outidx_hbm.at[wid].at[g], obuf.at[bg],
                                      semio).wait()

            @pl.when((l == 8) & (g < NGROUPS - 1))
            def _():
                pltpu.async_copy(inidx_hbm.at[wid].at[g + 1], ibuf.at[1 - bg],
                                 semii)
                pltpu.async_copy(outidx_hbm.at[wid].at[g + 1], obuf.at[1 - bg],
                                 semio)

            @pl.when(j >= NBUF)
            def _():
                pltpu.make_async_copy(rows_v.at[b],
                                      acc_sh.at[obuf.at[bg].at[l]],
                                      sems_[b]).wait()

            pltpu.async_copy(tf_hbm.at[ibuf.at[bg].at[l]], rows_v.at[b],
                             semg[b])

            b2 = (b + 3) % NBUF

            @pl.when(j >= 3)
            def _():
                j3 = j - 3
                g3 = j3 // G
                bg3 = g3 % 2
                l3 = j3 % G
                pltpu.make_async_copy(tf_hbm.at[ibuf.at[bg3].at[l3]],
                                      rows_v.at[b2], semg[b2]).wait()
                pltpu.async_copy(rows_v.at[b2],
                                 acc_sh.at[obuf.at[bg3].at[l3]],
                                 sems_[b2], add=True)
        return carry

    lax.fori_loop(0, N_CHUNKS // NBUF, step, 0)

    for e in range(3):
        ch = N_CHUNKS - 3 + e
        b2 = ch % NBUF
        g3 = ch // G
        pltpu.make_async_copy(tf_hbm.at[ibuf.at[g3 % 2].at[ch % G]],
                              rows_v.at[b2], semg[b2]).wait()
        pltpu.async_copy(rows_v.at[b2],
                         acc_sh.at[obuf.at[g3 % 2].at[ch % G]],
                         sems_[b2], add=True)
    for e in range(NBUF):
        ch = N_CHUNKS - NBUF + e
        b = ch % NBUF
        g3 = ch // G
        pltpu.make_async_copy(rows_v.at[b],
                              acc_sh.at[obuf.at[g3 % 2].at[ch % G]],
                              sems_[b]).wait()

    plsc.subcore_barrier()
    pltpu.sync_copy(acc_sh.at[pl.ds(base, ROWS_PT)],
                    out_hbm.at[c].at[pl.ds(base, ROWS_PT)])


_sc_gather_scatter = functools.partial(
    pl.kernel,
    out_type=jax.ShapeDtypeStruct((NC, N_PAD, COUT), jnp.float32),
    mesh=plsc.VectorSubcoreMesh(core_axis_name="c", subcore_axis_name="s",
                                num_cores=NC, num_subcores=NS),
    scratch_types=[
        pltpu.VMEM((2, G, CHUNK), jnp.int32),
        pltpu.VMEM((2, G, CHUNK), jnp.int32),
        pltpu.VMEM((NBUF, CHUNK, COUT), jnp.float32),
        pltpu.VMEM_SHARED((N_PAD, COUT), jnp.float32),
    ] + [pltpu.SemaphoreType.DMA] * (2 * NBUF + 2),
    compiler_params=pltpu.CompilerParams(use_tc_tiling_on_sc=False),
)(_sc_body)


def _comb_body(a_ref, o_ref):
    o_ref[...] = a_ref[0] + a_ref[1]


def _combine(acc128):
    nblk = (N_PAD // 8) // _CB_BLK
    return pl.pallas_call(
        _comb_body,
        grid=(nblk,),
        in_specs=[pl.BlockSpec((NC, _CB_BLK, 8 * COUT), lambda r: (0, r, 0))],
        out_specs=pl.BlockSpec((_CB_BLK, 8 * COUT), lambda r: (r, 0)),
        out_shape=jax.ShapeDtypeStruct((N_PAD // 8, 8 * COUT), jnp.float32),
    )(acc128)


def kernel(features, kernel, in_idx, out_idx):
    offs = (jnp.arange(K, dtype=jnp.int32) * NP8)[:, None]
    in_flat = (in_idx + offs).reshape(-1)
    out_flat = out_idx.reshape(-1)
    pad = TOTAL_PAD - TOTAL
    in_flat = jnp.concatenate([in_flat, jnp.zeros((pad,), jnp.int32)])
    out_flat = jnp.concatenate([out_flat, jnp.full((pad,), N, jnp.int32)])
    in_resh = in_flat.reshape(NW, NGROUPS, G, CHUNK)
    out_resh = out_flat.reshape(NW, NGROUPS, G, CHUNK)

    features8 = jnp.pad(features, ((0, NP8 - N), (0, 0))).reshape(R8, 8 * CIN)
    wbd = jnp.einsum('qr,kcd->kqcrd', jnp.eye(8, dtype=jnp.float32),
                     kernel).reshape(K, 8 * CIN, 8 * COUT)

    tf = _transform(features8, wbd).reshape(K * NP8, COUT)
    acc = _sc_gather_scatter(tf, in_resh, out_resh)
    out128 = _combine(acc.reshape(NC, N_PAD // 8, 8 * COUT))
    return out128.reshape(N_PAD, COUT)[:N]
